# Initial kernel scaffold; baseline (speedup 1.0000x reference)
import jax
import jax.numpy as jnp
from jax.experimental import pallas as pl


def kernel(x):
    raise NotImplementedError("write your pallas kernel here")


# speedup vs baseline: 590.4639x; 590.4639x over previous
import jax
import jax.numpy as jnp
from jax import lax
from jax.experimental import pallas as pl
from jax.experimental.pallas import tpu as pltpu

_W = 4096
_G = _W // 128
_OUT = (_W - 4) // 2 + 1
_R = 256


def _npool_block(x_ref, out_ref):
    x = x_ref[...]
    r = x.shape[0]

    lane = lax.broadcasted_iota(jnp.int32, (r, 128), 1)
    idx = jnp.where(lane < 64, 2 * lane, 2 * (lane - 64) + 1)
    lo = lane < 64

    A = [
        jnp.take_along_axis(x[:, 128 * g:128 * (g + 1)], idx, axis=1)
        for g in range(_G)
    ]
    E_parts = []
    O_parts = []
    for t in range(_G // 2):
        a0 = A[2 * t]
        a1 = A[2 * t + 1]
        r0 = pltpu.roll(a0, 64, axis=1)
        r1 = pltpu.roll(a1, 64, axis=1)
        E_parts.append(jnp.where(lo, a0, r1))
        O_parts.append(jnp.where(lo, r0, a1))
    E = jnp.concatenate(E_parts, axis=1)
    O = jnp.concatenate(O_parts, axis=1)

    v0 = E[:, :_OUT]
    v1 = O[:, :_OUT]
    v2 = E[:, 1:_OUT + 1]
    v3 = O[:, 1:_OUT + 1]

    ge01 = v0 >= v1
    ge23 = v2 >= v3
    m01 = jnp.maximum(v0, v1)
    m23 = jnp.maximum(v2, v3)
    ge = m01 >= m23
    m = jnp.maximum(m01, m23)

    s = (v0 + v1) + (v2 + v3)
    z = jnp.where(ge, v3, v0)
    interior = 0.25 * (m + s - z)
    edge = (ge & ge01) | (~ge & ~ge23)
    out_ref[...] = jnp.where(edge, m, interior)


def kernel(x):
    b, c, h, w = x.shape
    rows = b * c * h
    xf = x.reshape(rows, w)
    grid = (rows // _R,)
    out = pl.pallas_call(
        _npool_block,
        out_shape=jax.ShapeDtypeStruct((rows, _OUT), x.dtype),
        grid=grid,
        in_specs=[pl.BlockSpec((_R, w), lambda i: (i, 0))],
        out_specs=pl.BlockSpec((_R, _OUT), lambda i: (i, 0)),
        compiler_params=pltpu.CompilerParams(
            dimension_semantics=("parallel",),
        ),
        name="npool_fused",
    )(xf)
    return out.reshape(b, c, h, _OUT)


# speedup vs baseline: 755.9673x; 1.2803x over previous
import jax
import jax.numpy as jnp
from jax import lax
from jax.experimental import pallas as pl
from jax.experimental.pallas import tpu as pltpu

_W = 4096
_P = _W // 256
_OUT = (_W - 4) // 2 + 1
_R = 256


def _npool_block(x_ref, out_ref):
    x = x_ref[...]
    r = x.shape[0]

    lane = lax.broadcasted_iota(jnp.int32, (r, 128), 1)
    idx = jnp.where(lane < 64, 2 * lane, 2 * (lane - 64) + 1)
    lo = lane < 64
    last = lane == 127

    for t in range(_P):
        s0 = x[:, 256 * t:256 * t + 128]
        s1 = x[:, 256 * t + 128:256 * t + 256]
        a0 = jnp.take_along_axis(s0, idx, axis=1)
        a1 = jnp.take_along_axis(s1, idx, axis=1)
        E = jnp.where(lo, a0, pltpu.roll(a1, 64, axis=1))
        O = jnp.where(lo, pltpu.roll(a0, 64, axis=1), a1)

        v2 = pltpu.roll(E, -1, axis=1)
        v3 = pltpu.roll(O, -1, axis=1)
        if t < _P - 1:
            nxt = x[:, 256 * t + 256:256 * t + 384]
            v2 = jnp.where(last, pltpu.roll(nxt, 127, axis=1), v2)
            v3 = jnp.where(last, pltpu.roll(nxt, 126, axis=1), v3)

        ge01 = E >= O
        ge23 = v2 >= v3
        m01 = jnp.maximum(E, O)
        m23 = jnp.maximum(v2, v3)
        ge = m01 >= m23
        m = jnp.maximum(m01, m23)

        s = (E + O) + (v2 + v3)
        z = jnp.where(ge, v3, E)
        interior = 0.25 * (m + s - z)
        edge = (ge & ge01) | (~ge & ~ge23)
        out = jnp.where(edge, m, interior)

        if t < _P - 1:
            out_ref[:, 128 * t:128 * t + 128] = out
        else:
            out_ref[:, 128 * t:_OUT] = out[:, :127]


def kernel(x):
    b, c, h, w = x.shape
    rows = b * c * h
    xf = x.reshape(rows, w)
    grid = (rows // _R,)
    out = pl.pallas_call(
        _npool_block,
        out_shape=jax.ShapeDtypeStruct((rows, _OUT), x.dtype),
        grid=grid,
        in_specs=[pl.BlockSpec((_R, w), lambda i: (i, 0))],
        out_specs=pl.BlockSpec((_R, _OUT), lambda i: (i, 0)),
        compiler_params=pltpu.CompilerParams(
            dimension_semantics=("parallel",),
        ),
        name="npool_fused",
    )(xf)
    return out.reshape(b, c, h, _OUT)


# speedup vs baseline: 1178.0456x; 1.5583x over previous
import jax
import jax.numpy as jnp
from jax import lax
from jax.experimental import pallas as pl
from jax.experimental.pallas import tpu as pltpu

_W = 4096
_P = _W // 256
_OUT = (_W - 4) // 2 + 1
_R = 256
_CHUNK = 2


def _npool_block(x_ref, out_ref):
    r = x_ref.shape[0]

    lane = lax.broadcasted_iota(jnp.int32, (r, 128), 1)
    idx_a = jnp.where(lane < 64, 2 * lane, 2 * (lane - 64) + 1)
    idx_b = jnp.where(lane < 64, 2 * lane + 1, 2 * (lane - 64))
    lo = lane < 64
    last = lane == 127

    ee_carry = None
    ro_carry = None
    for c in range(_P // _CHUNK - 1, -1, -1):
        ts = [c * _CHUNK + u for u in range(_CHUNK - 1, -1, -1)]
        a0s = [
            jnp.take_along_axis(x_ref[:, 256 * t:256 * t + 128], idx_a, axis=1)
            for t in ts
        ]
        a1s = [
            jnp.take_along_axis(x_ref[:, 256 * t + 128:256 * t + 256], idx_b, axis=1)
            for t in ts
        ]
        Es = [jnp.where(lo, a0, a1) for a0, a1 in zip(a0s, a1s)]
        O2s = [jnp.where(lo, a1, a0) for a0, a1 in zip(a0s, a1s)]
        Os = [pltpu.roll(o2, 64, axis=1) for o2 in O2s]
        ees = [pltpu.roll(e, 127, axis=1) for e in Es]
        ros = [pltpu.roll(o2, 63, axis=1) for o2 in O2s]
        for u, t in enumerate(ts):
            E, O, ee, ro = Es[u], Os[u], ees[u], ros[u]
            if t == _P - 1:
                v2, v3 = ee, ro
            else:
                v2 = jnp.where(last, ee_carry, ee)
                v3 = jnp.where(last, ro_carry, ro)
            ee_carry, ro_carry = ee, ro

            ge01 = E >= O
            ge23 = v2 >= v3
            m01 = jnp.maximum(E, O)
            m23 = jnp.maximum(v2, v3)
            ge = m01 >= m23

            q1 = 0.25 * (E + v2) + 0.5 * O
            q2 = 0.25 * (O + v3) + 0.5 * v2
            out = jnp.where(
                ge, jnp.where(ge01, E, q1), jnp.where(ge23, q2, v3)
            )

            if t < _P - 1:
                out_ref[:, 128 * t:128 * t + 128] = out
            else:
                out_ref[:, 128 * t:_OUT] = out[:, :127]


def kernel(x):
    b, c, h, w = x.shape
    rows = b * c * h
    xf = x.reshape(rows, w)
    grid = (rows // _R,)
    out = pl.pallas_call(
        _npool_block,
        out_shape=jax.ShapeDtypeStruct((rows, _OUT), x.dtype),
        grid=grid,
        in_specs=[pl.BlockSpec((_R, w), lambda i: (i, 0))],
        out_specs=pl.BlockSpec((_R, _OUT), lambda i: (i, 0)),
        compiler_params=pltpu.CompilerParams(
            dimension_semantics=("parallel",),
        ),
        name="npool_fused",
    )(xf)
    return out.reshape(b, c, h, _OUT)
